# baseline (speedup 1.0000x reference)
import functools

import jax
import jax.numpy as jnp
from jax import lax
from jax.experimental import pallas as pl
from jax.experimental.pallas import tpu as pltpu
from jax.experimental.pallas import tpu_sc as plsc

V = 1000000
D = 64
S = 16384
W = 50
NW = 32
SB = S // 128
SB_W = SB // NW
NB = SB_W * W

_mesh = plsc.VectorSubcoreMesh(core_axis_name="c", subcore_axis_name="s")


@functools.partial(
    pl.kernel,
    out_type=jax.ShapeDtypeStruct((W, D, S), jnp.float32),
    mesh=_mesh,
    scratch_types=[
        pltpu.VMEM((2, 128), jnp.int32),
        pltpu.VMEM((2, 128), jnp.int32),
        pltpu.VMEM((2, 128), jnp.int32),
        pltpu.VMEM((2, 128, 128), jnp.float32),
        pltpu.VMEM((2, D, 128), jnp.float32),
        pltpu.SemaphoreType.DMA((2,)),
        pltpu.SemaphoreType.DMA((2,)),
        pltpu.SemaphoreType.DMA((2,)),
    ],
    compiler_params=pltpu.CompilerParams(
        use_tc_tiling_on_sc=True, needs_layout_passes=False
    ),
)
def _emb(idx_hbm, table_hbm, out_hbm, idxb, jdx, pcol, gbuf, tbuf,
         sem_g, sem_i, sem_w):
    wid = lax.axis_index("s") * 2 + lax.axis_index("c")
    sblock0 = wid * SB_W

    def idx_off(b):
        sb = b // W
        w = b - sb * W
        return pl.multiple_of(w * S + (sblock0 + sb) * 128, 128), w

    def fire_idx(b, s):
        off, _ = idx_off(b)
        pltpu.async_copy(idx_hbm.at[pl.ds(off, 128)], idxb.at[s], sem_i.at[s])

    def wait_idx(b, s):
        off, _ = idx_off(b)
        pltpu.make_async_copy(
            idx_hbm.at[pl.ds(off, 128)], idxb.at[s], sem_i.at[s]
        ).wait()

    def compute_jdx(s):
        for l0 in range(0, 128, 16):
            ids = idxb[s, pl.ds(l0, 16)]
            jdx[s, pl.ds(l0, 16)] = lax.shift_right_logical(ids, 1)
            pcol[s, pl.ds(l0, 16)] = (ids & jnp.int32(1)) * jnp.int32(D)

    def fire_gather(s):
        pltpu.async_copy(table_hbm.at[jdx.at[s]], gbuf.at[s], sem_g.at[s])

    def wait_gather(s):
        pltpu.make_async_copy(
            table_hbm.at[jdx.at[s]], gbuf.at[s], sem_g.at[s]
        ).wait()

    def transpose(s):
        def dstep(d, carry):
            for l0 in range(0, 128, 16):
                rows = lax.iota(jnp.int32, 16) + jnp.int32(l0)
                cols = pcol[s, pl.ds(l0, 16)] + d
                tbuf[s, d, pl.ds(l0, 16)] = plsc.load_gather(
                    gbuf.at[s], [rows, cols]
                )
            return carry

        lax.fori_loop(0, D, dstep, 0)

    def fire_wb(b, s):
        off, w = idx_off(b)
        s0 = pl.multiple_of(off - w * S, 128)
        pltpu.async_copy(
            tbuf.at[s], out_hbm.at[w, :, pl.ds(s0, 128)], sem_w.at[s]
        )

    def wait_wb(b, s):
        off, w = idx_off(b)
        s0 = pl.multiple_of(off - w * S, 128)
        pltpu.make_async_copy(
            tbuf.at[s], out_hbm.at[w, :, pl.ds(s0, 128)], sem_w.at[s]
        ).wait()

    fire_idx(0, 0)
    wait_idx(0, 0)
    compute_jdx(0)
    fire_gather(0)
    fire_idx(1, 1)

    def step(b, carry):
        s = lax.rem(b, 2)
        ns = lax.rem(b + 1, 2)
        wait_gather(s)

        @pl.when(b + 1 < NB)
        def _():
            wait_idx(b + 1, ns)
            compute_jdx(ns)

        @pl.when(b + 2 < NB)
        def _():
            fire_idx(b + 2, s)

        @pl.when(b + 1 < NB)
        def _():
            fire_gather(ns)

        @pl.when(b >= 2)
        def _():
            wait_wb(b - 2, s)

        transpose(s)
        fire_wb(b, s)
        return carry

    lax.fori_loop(0, NB, step, 0)
    wait_wb(NB - 2, (NB - 2) % 2)
    wait_wb(NB - 1, (NB - 1) % 2)


def kernel(input_, weight):
    idx_flat = jnp.reshape(jnp.transpose(input_), (S * W,))
    table2 = jnp.reshape(weight, (V // 2, 2 * D))
    y = _emb(idx_flat, table2)
    return jnp.transpose(y, (2, 0, 1))


# speedup vs baseline: 2.2367x; 2.2367x over previous
import functools

import jax
import jax.numpy as jnp
from jax import lax
from jax.experimental import pallas as pl
from jax.experimental.pallas import tpu as pltpu
from jax.experimental.pallas import tpu_sc as plsc

V = 1000000
D = 64
S = 16384
W = 50
NW = 32
SB = S // 128
SB_W = SB // NW
NB = SB_W * W

_mesh = plsc.VectorSubcoreMesh(core_axis_name="c", subcore_axis_name="s")


@functools.partial(
    pl.kernel,
    out_type=jax.ShapeDtypeStruct((W, D, S), jnp.float32),
    mesh=_mesh,
    scratch_types=[
        pltpu.VMEM((2, 128), jnp.int32),
        pltpu.VMEM((2, 128), jnp.int32),
        pltpu.VMEM((2, 128), jnp.int32),
        pltpu.VMEM((2, 128, 128), jnp.float32),
        pltpu.VMEM((2, D, 128), jnp.float32),
        pltpu.SemaphoreType.DMA((2,)),
        pltpu.SemaphoreType.DMA((2,)),
        pltpu.SemaphoreType.DMA((2,)),
    ],
    compiler_params=pltpu.CompilerParams(
        use_tc_tiling_on_sc=True, needs_layout_passes=False
    ),
)
def _emb(idx_hbm, table_hbm, out_hbm, idxb, jdx, pcol, gbuf, tbuf,
         sem_g, sem_i, sem_w):
    wid = lax.axis_index("s") * 2 + lax.axis_index("c")
    sblock0 = wid * SB_W

    def idx_off(b):
        sb = b // W
        w = b - sb * W
        return pl.multiple_of(w * S + (sblock0 + sb) * 128, 128), w

    def fire_idx(b, s):
        off, _ = idx_off(b)
        pltpu.async_copy(idx_hbm.at[pl.ds(off, 128)], idxb.at[s], sem_i.at[s])

    def wait_idx(b, s):
        off, _ = idx_off(b)
        pltpu.make_async_copy(
            idx_hbm.at[pl.ds(off, 128)], idxb.at[s], sem_i.at[s]
        ).wait()

    def compute_jdx(s):
        for l0 in range(0, 128, 16):
            ids = idxb[s, pl.ds(l0, 16)]
            jdx[s, pl.ds(l0, 16)] = lax.shift_right_logical(ids, 1)
            pcol[s, pl.ds(l0, 16)] = (ids & jnp.int32(1)) * jnp.int32(D)

    def fire_gather(s):
        pltpu.async_copy(table_hbm.at[jdx.at[s]], gbuf.at[s], sem_g.at[s])

    def wait_gather(s):
        pltpu.make_async_copy(
            table_hbm.at[jdx.at[s]], gbuf.at[s], sem_g.at[s]
        ).wait()

    def transpose(s):
        lanes = lax.iota(jnp.int32, 16)

        def jstep(j, carry):
            rot = (lanes + j) & jnp.int32(15)
            for l0 in range(0, 128, 16):
                pcolv = pcol[s, pl.ds(l0, 16)]
                rowsrc = lanes + jnp.int32(l0)
                for d0 in range(0, D, 16):
                    rd = rot + jnp.int32(d0)
                    x = plsc.load_gather(gbuf.at[s], [rowsrc, pcolv + rd])
                    plsc.store_scatter(tbuf.at[s], [rd, rowsrc], x)
            return carry

        lax.fori_loop(0, 16, jstep, 0)

    def fire_wb(b, s):
        off, w = idx_off(b)
        s0 = pl.multiple_of(off - w * S, 128)
        pltpu.async_copy(
            tbuf.at[s], out_hbm.at[w, :, pl.ds(s0, 128)], sem_w.at[s]
        )

    def wait_wb(b, s):
        off, w = idx_off(b)
        s0 = pl.multiple_of(off - w * S, 128)
        pltpu.make_async_copy(
            tbuf.at[s], out_hbm.at[w, :, pl.ds(s0, 128)], sem_w.at[s]
        ).wait()

    fire_idx(0, 0)
    wait_idx(0, 0)
    compute_jdx(0)
    fire_gather(0)
    fire_idx(1, 1)

    def step(b, carry):
        s = lax.rem(b, 2)
        ns = lax.rem(b + 1, 2)
        wait_gather(s)

        @pl.when(b + 1 < NB)
        def _():
            wait_idx(b + 1, ns)
            compute_jdx(ns)

        @pl.when(b + 2 < NB)
        def _():
            fire_idx(b + 2, s)

        @pl.when(b + 1 < NB)
        def _():
            fire_gather(ns)

        @pl.when(b >= 2)
        def _():
            wait_wb(b - 2, s)

        transpose(s)
        fire_wb(b, s)
        return carry

    lax.fori_loop(0, NB, step, 0)
    wait_wb(NB - 2, (NB - 2) % 2)
    wait_wb(NB - 1, (NB - 1) % 2)


def kernel(input_, weight):
    idx_flat = jnp.reshape(jnp.transpose(input_), (S * W,))
    table2 = jnp.reshape(weight, (V // 2, 2 * D))
    y = _emb(idx_flat, table2)
    return jnp.transpose(y, (2, 0, 1))


# speedup vs baseline: 2.2372x; 1.0002x over previous
import functools

import jax
import jax.numpy as jnp
from jax import lax
from jax.experimental import pallas as pl
from jax.experimental.pallas import tpu as pltpu
from jax.experimental.pallas import tpu_sc as plsc

V = 1000000
D = 64
S = 16384
W = 50
NW = 32
SB = S // 128
SB_W = SB // NW
NB = SB_W * W

_mesh = plsc.VectorSubcoreMesh(core_axis_name="c", subcore_axis_name="s")


@functools.partial(
    pl.kernel,
    out_type=jax.ShapeDtypeStruct((W, D, S), jnp.float32),
    mesh=_mesh,
    scratch_types=[
        pltpu.VMEM((4, 128), jnp.int32),
        pltpu.VMEM((4, 128), jnp.int32),
        pltpu.VMEM((4, 128), jnp.int32),
        pltpu.VMEM((4, 128, 128), jnp.float32),
        pltpu.VMEM((2, D, 128), jnp.float32),
        pltpu.SemaphoreType.DMA((4,)),
        pltpu.SemaphoreType.DMA((4,)),
        pltpu.SemaphoreType.DMA((2,)),
    ],
    compiler_params=pltpu.CompilerParams(
        use_tc_tiling_on_sc=True, needs_layout_passes=False
    ),
)
def _emb(idx_hbm, table_hbm, out_hbm, idxb, jdx, pcol, gbuf, tbuf,
         sem_g, sem_i, sem_w):
    wid = lax.axis_index("s") * 2 + lax.axis_index("c")
    sblock0 = wid * SB_W

    def idx_off(b):
        sb = b // W
        w = b - sb * W
        return pl.multiple_of(w * S + (sblock0 + sb) * 128, 128), w

    def fire_idx(b, s):
        off, _ = idx_off(b)
        pltpu.async_copy(idx_hbm.at[pl.ds(off, 128)], idxb.at[s], sem_i.at[s])

    def wait_idx(b, s):
        off, _ = idx_off(b)
        pltpu.make_async_copy(
            idx_hbm.at[pl.ds(off, 128)], idxb.at[s], sem_i.at[s]
        ).wait()

    def compute_jdx(s):
        for l0 in range(0, 128, 16):
            ids = idxb[s, pl.ds(l0, 16)]
            jdx[s, pl.ds(l0, 16)] = lax.shift_right_logical(ids, 1)
            pcol[s, pl.ds(l0, 16)] = (ids & jnp.int32(1)) * jnp.int32(D)

    def fire_gather(s):
        pltpu.async_copy(table_hbm.at[jdx.at[s]], gbuf.at[s], sem_g.at[s])

    def wait_gather(s):
        pltpu.make_async_copy(
            table_hbm.at[jdx.at[s]], gbuf.at[s], sem_g.at[s]
        ).wait()

    def transpose(s4, s2):
        lanes = lax.iota(jnp.int32, 16)

        def jstep(j, carry):
            rot = (lanes + j) & jnp.int32(15)
            for l0 in range(0, 128, 16):
                pcolv = pcol[s4, pl.ds(l0, 16)]
                rowsrc = lanes + jnp.int32(l0)
                for d0 in range(0, D, 16):
                    rd = rot + jnp.int32(d0)
                    x = plsc.load_gather(gbuf.at[s4], [rowsrc, pcolv + rd])
                    plsc.store_scatter(tbuf.at[s2], [rd, rowsrc], x)
            return carry

        lax.fori_loop(0, 16, jstep, 0)

    def fire_wb(b, s):
        off, w = idx_off(b)
        s0 = pl.multiple_of(off - w * S, 128)
        pltpu.async_copy(
            tbuf.at[s], out_hbm.at[w, :, pl.ds(s0, 128)], sem_w.at[s]
        )

    def wait_wb(b, s):
        off, w = idx_off(b)
        s0 = pl.multiple_of(off - w * S, 128)
        pltpu.make_async_copy(
            tbuf.at[s], out_hbm.at[w, :, pl.ds(s0, 128)], sem_w.at[s]
        ).wait()

    fire_idx(0, 0)
    fire_idx(1, 1)
    fire_idx(2, 2)
    for b0 in range(3):
        wait_idx(b0, b0)
        compute_jdx(b0)
        fire_gather(b0)
    fire_idx(3, 3)

    def step(b, carry):
        s4 = lax.rem(b, 4)
        s2 = lax.rem(b, 2)
        wait_gather(s4)

        @pl.when(b + 3 < NB)
        def _():
            ns = lax.rem(b + 3, 4)
            wait_idx(b + 3, ns)
            compute_jdx(ns)
            fire_gather(ns)

        @pl.when(b + 4 < NB)
        def _():
            fire_idx(b + 4, s4)

        @pl.when(b >= 2)
        def _():
            wait_wb(b - 2, s2)

        transpose(s4, s2)
        fire_wb(b, s2)
        return carry

    lax.fori_loop(0, NB, step, 0)
    wait_wb(NB - 2, (NB - 2) % 2)
    wait_wb(NB - 1, (NB - 1) % 2)


def kernel(input_, weight):
    idx_flat = jnp.reshape(jnp.transpose(input_), (S * W,))
    table2 = jnp.reshape(weight, (V // 2, 2 * D))
    y = _emb(idx_flat, table2)
    return jnp.transpose(y, (2, 0, 1))


# speedup vs baseline: 2.2403x; 1.0014x over previous
import functools

import jax
import jax.numpy as jnp
from jax import lax
from jax.experimental import pallas as pl
from jax.experimental.pallas import tpu as pltpu
from jax.experimental.pallas import tpu_sc as plsc

V = 1000000
D = 64
S = 16384
W = 50
NW = 32
SB = S // 128
SB_W = SB // NW
NB = SB_W * W

_mesh = plsc.VectorSubcoreMesh(core_axis_name="c", subcore_axis_name="s")


@functools.partial(
    pl.kernel,
    out_type=jax.ShapeDtypeStruct((W, D, S), jnp.float32),
    mesh=_mesh,
    scratch_types=[
        pltpu.VMEM((4, 128), jnp.int32),
        pltpu.VMEM((4, 128), jnp.int32),
        pltpu.VMEM((4, 128), jnp.int32),
        pltpu.VMEM((4, 128, 128), jnp.float32),
        pltpu.VMEM((2, D, 128), jnp.float32),
        pltpu.SemaphoreType.DMA((4,)),
        pltpu.SemaphoreType.DMA((4,)),
        pltpu.SemaphoreType.DMA((2,)),
    ],
    compiler_params=pltpu.CompilerParams(
        use_tc_tiling_on_sc=True, needs_layout_passes=False
    ),
)
def _emb(idx_hbm, table_hbm, out_hbm, idxb, jdx, pcol, gbuf, tbuf,
         sem_g, sem_i, sem_w):
    wid = lax.axis_index("s") * 2 + lax.axis_index("c")
    sblock0 = wid * SB_W

    def idx_off(b):
        sb = b // W
        w = b - sb * W
        return pl.multiple_of(w * S + (sblock0 + sb) * 128, 128), w

    def fire_idx(b, s):
        off, _ = idx_off(b)
        pltpu.async_copy(idx_hbm.at[pl.ds(off, 128)], idxb.at[s], sem_i.at[s])

    def wait_idx(b, s):
        off, _ = idx_off(b)
        pltpu.make_async_copy(
            idx_hbm.at[pl.ds(off, 128)], idxb.at[s], sem_i.at[s]
        ).wait()

    def compute_jdx(s):
        for l0 in range(0, 128, 16):
            ids = idxb[s, pl.ds(l0, 16)]
            jdx[s, pl.ds(l0, 16)] = lax.shift_right_logical(ids, 1)
            pcol[s, pl.ds(l0, 16)] = (ids & jnp.int32(1)) * jnp.int32(D)

    def fire_gather(s):
        pltpu.async_copy(table_hbm.at[jdx.at[s]], gbuf.at[s], sem_g.at[s])

    def wait_gather(s):
        pltpu.make_async_copy(
            table_hbm.at[jdx.at[s]], gbuf.at[s], sem_g.at[s]
        ).wait()

    def transpose(s4, s2):
        lanes = lax.iota(jnp.int32, 16)

        def jstep(j, carry):
            rot = (lanes + j) & jnp.int32(15)
            rds = [rot + jnp.int32(d0) for d0 in range(0, D, 16)]
            for l0 in range(0, 128, 16):
                pcolv = pcol[s4, pl.ds(l0, 16)]
                rowsrc = lanes + jnp.int32(l0)
                for di in range(D // 16):
                    x = plsc.load_gather(
                        gbuf.at[s4], [rowsrc, pcolv + rds[di]]
                    )
                    plsc.store_scatter(tbuf.at[s2], [rds[di], rowsrc], x)
            return carry

        lax.fori_loop(0, 16, jstep, 0)

    def fire_wb(b, s):
        off, w = idx_off(b)
        s0 = pl.multiple_of(off - w * S, 128)
        pltpu.async_copy(
            tbuf.at[s], out_hbm.at[w, :, pl.ds(s0, 128)], sem_w.at[s]
        )

    def wait_wb(b, s):
        off, w = idx_off(b)
        s0 = pl.multiple_of(off - w * S, 128)
        pltpu.make_async_copy(
            tbuf.at[s], out_hbm.at[w, :, pl.ds(s0, 128)], sem_w.at[s]
        ).wait()

    fire_idx(0, 0)
    fire_idx(1, 1)
    fire_idx(2, 2)
    for b0 in range(3):
        wait_idx(b0, b0)
        compute_jdx(b0)
        fire_gather(b0)
    fire_idx(3, 3)

    def step(b, carry):
        s4 = lax.rem(b, 4)
        s2 = lax.rem(b, 2)
        wait_gather(s4)

        @pl.when(b + 3 < NB)
        def _():
            ns = lax.rem(b + 3, 4)
            wait_idx(b + 3, ns)
            compute_jdx(ns)
            fire_gather(ns)

        @pl.when(b + 4 < NB)
        def _():
            fire_idx(b + 4, s4)

        @pl.when(b >= 2)
        def _():
            wait_wb(b - 2, s2)

        transpose(s4, s2)
        fire_wb(b, s2)
        return carry

    lax.fori_loop(0, NB, step, 0)
    wait_wb(NB - 2, (NB - 2) % 2)
    wait_wb(NB - 1, (NB - 1) % 2)


def kernel(input_, weight):
    idx_flat = jnp.reshape(jnp.transpose(input_), (S * W,))
    table2 = jnp.reshape(weight, (V // 2, 2 * D))
    y = _emb(idx_flat, table2)
    return jnp.transpose(y, (2, 0, 1))


# speedup vs baseline: 2.4819x; 1.1079x over previous
import functools

import jax
import jax.numpy as jnp
from jax import lax
from jax.experimental import pallas as pl
from jax.experimental.pallas import tpu as pltpu
from jax.experimental.pallas import tpu_sc as plsc

V = 1000000
D = 64
S = 16384
W = 50
NW = 32
SB = S // 128
SB_W = SB // NW
NB = SB_W * W

_mesh = plsc.VectorSubcoreMesh(core_axis_name="c", subcore_axis_name="s")


@functools.partial(
    pl.kernel,
    out_type=jax.ShapeDtypeStruct((W, D, S), jnp.float32),
    mesh=_mesh,
    scratch_types=[
        pltpu.VMEM((4, 128), jnp.int32),
        pltpu.VMEM((4, 128), jnp.int32),
        pltpu.VMEM((4, 128), jnp.int32),
        pltpu.VMEM((4, 128, 128), jnp.float32),
        pltpu.VMEM((2, D, 128), jnp.float32),
        pltpu.SemaphoreType.DMA((4,)),
        pltpu.SemaphoreType.DMA((4,)),
        pltpu.SemaphoreType.DMA((2,)),
    ],
    compiler_params=pltpu.CompilerParams(
        use_tc_tiling_on_sc=True, needs_layout_passes=False
    ),
)
def _emb(idx_hbm, table_hbm, out_hbm, idxb, jdx, pcol, gbuf, tbuf,
         sem_g, sem_i, sem_w):
    wid = lax.axis_index("s") * 2 + lax.axis_index("c")
    sblock0 = wid * SB_W

    def idx_off(b):
        sb = b // W
        w = b - sb * W
        return pl.multiple_of(w * S + (sblock0 + sb) * 128, 128), w

    def fire_idx(b, s):
        off, _ = idx_off(b)
        pltpu.async_copy(idx_hbm.at[pl.ds(off, 128)], idxb.at[s], sem_i.at[s])

    def wait_idx(b, s):
        off, _ = idx_off(b)
        pltpu.make_async_copy(
            idx_hbm.at[pl.ds(off, 128)], idxb.at[s], sem_i.at[s]
        ).wait()

    def compute_jdx(s):
        for l0 in range(0, 128, 16):
            ids = idxb[s, pl.ds(l0, 16)]
            jdx[s, pl.ds(l0, 16)] = lax.shift_right_logical(ids, 1)
            pcol[s, pl.ds(l0, 16)] = (ids & jnp.int32(1)) * jnp.int32(D)

    def fire_gather(s):
        pltpu.async_copy(table_hbm.at[jdx.at[s]], gbuf.at[s], sem_g.at[s])

    def wait_gather(s):
        pltpu.make_async_copy(
            table_hbm.at[jdx.at[s]], gbuf.at[s], sem_g.at[s]
        ).wait()

    def transpose(s4, s2):
        lanes = lax.iota(jnp.int32, 16)

        def jstep(j, carry):
            rot = (lanes + j) & jnp.int32(15)
            rds = [rot + jnp.int32(d0) for d0 in range(0, D, 16)]
            for l0 in range(0, 128, 16):
                pcolv = pcol[s4, pl.ds(l0, 16)]
                rowsrc = lanes + jnp.int32(l0)
                for di in range(D // 16):
                    x = plsc.load_gather(
                        gbuf.at[s4], [rowsrc, pcolv + rds[di]]
                    )
                    plsc.store_scatter(tbuf.at[s2], [rds[di], rowsrc], x)
            return carry

        lax.fori_loop(0, 16, jstep, 0)

    def fire_wb(b, s):
        off, w = idx_off(b)
        s0 = pl.multiple_of(off - w * S, 128)
        pltpu.async_copy(
            tbuf.at[s], out_hbm.at[w, :, pl.ds(s0, 128)], sem_w.at[s]
        )

    def wait_wb(b, s):
        off, w = idx_off(b)
        s0 = pl.multiple_of(off - w * S, 128)
        pltpu.make_async_copy(
            tbuf.at[s], out_hbm.at[w, :, pl.ds(s0, 128)], sem_w.at[s]
        ).wait()

    fire_idx(0, 0)
    fire_idx(1, 1)
    fire_idx(2, 2)
    for b0 in range(3):
        wait_idx(b0, b0)
        compute_jdx(b0)
        fire_gather(b0)
    fire_idx(3, 3)

    def step(b, carry):
        s4 = lax.rem(b, 4)
        s2 = lax.rem(b, 2)
        wait_gather(s4)

        @pl.when(b + 3 < NB)
        def _():
            ns = lax.rem(b + 3, 4)
            wait_idx(b + 3, ns)
            compute_jdx(ns)
            fire_gather(ns)

        @pl.when(b + 4 < NB)
        def _():
            fire_idx(b + 4, s4)

        @pl.when(b >= 2)
        def _():
            wait_wb(b - 2, s2)

        transpose(s4, s2)
        fire_wb(b, s2)
        return carry

    lax.fori_loop(0, NB, step, 0)
    wait_wb(NB - 2, (NB - 2) % 2)
    wait_wb(NB - 1, (NB - 1) % 2)


NFB = V // 128


@functools.partial(
    pl.kernel,
    out_type=jax.ShapeDtypeStruct((V // 2, 2 * D), jnp.float32),
    mesh=_mesh,
    scratch_types=[
        pltpu.VMEM((2, D, 128), jnp.float32),
        pltpu.VMEM((2, D, 128), jnp.float32),
        pltpu.VMEM((D, D), jnp.float32),
        pltpu.VMEM((32, 2 * D), jnp.float32),
        pltpu.SemaphoreType.DMA((2,)),
        pltpu.SemaphoreType.DMA((2,)),
    ],
    compiler_params=pltpu.CompilerParams(
        use_tc_tiling_on_sc=True, needs_layout_passes=False
    ),
)
def _pairs(wt_hbm, tail_hbm, out_hbm, xb, tb, xt, tt, sem_i, sem_o):
    wid = lax.axis_index("s") * 2 + lax.axis_index("c")
    lanes = lax.iota(jnp.int32, 16)

    def blk(i):
        return i * NW + wid

    def fire_in(i, s):
        g = pl.multiple_of(blk(i) * 128, 128)
        pltpu.async_copy(wt_hbm.at[:, pl.ds(g, 128)], xb.at[s], sem_i.at[s])

    def wait_in(i, s):
        g = pl.multiple_of(blk(i) * 128, 128)
        pltpu.make_async_copy(
            wt_hbm.at[:, pl.ds(g, 128)], xb.at[s], sem_i.at[s]
        ).wait()

    def fire_out(i, s):
        r = pl.multiple_of(blk(i) * 64, 64)
        pltpu.async_copy(tb.at[s], out_hbm.at[pl.ds(r, 64)], sem_o.at[s])

    def wait_out(i, s):
        r = pl.multiple_of(blk(i) * 64, 64)
        pltpu.make_async_copy(
            tb.at[s], out_hbm.at[pl.ds(r, 64)], sem_o.at[s]
        ).wait()

    def tblock(src, dst, nt):
        half = (lanes & jnp.int32(1)) * jnp.int32(D)
        kh = lax.shift_right_logical(lanes, 1)

        def jstep(j, carry):
            rot = (lanes + j) & jnp.int32(15)
            srows = [rot + jnp.int32(d0) for d0 in range(0, D, 16)]
            dcols = [half + sr for sr in srows]
            for t0 in range(0, nt, 16):
                scols = lanes + jnp.int32(t0)
                drows = kh + jnp.int32(t0 // 2)
                for di in range(D // 16):
                    x = plsc.load_gather(src, [srows[di], scols])
                    plsc.store_scatter(dst, [drows, dcols[di]], x)
            return carry

        lax.fori_loop(0, 16, jstep, 0)

    NI = (NFB + NW - 1) // NW

    fire_in(0, 0)
    fire_in(1, 1)

    def step(i, carry):
        s = lax.rem(i, 2)

        @pl.when(blk(i) < NFB)
        def _():
            wait_in(i, s)

            @pl.when(i >= 2)
            def _():
                wait_out(i - 2, s)

            tblock(xb.at[s], tb.at[s], 128)
            fire_out(i, s)

            @pl.when(blk(i + 2) < NFB)
            def _():
                fire_in(i + 2, s)

        return carry

    lax.fori_loop(0, NI, step, 0)

    @pl.when(blk(NI - 2) < NFB)
    def _():
        wait_out(NI - 2, (NI - 2) % 2)

    @pl.when(blk(NI - 1) < NFB)
    def _():
        wait_out(NI - 1, (NI - 1) % 2)

    @pl.when(wid == 0)
    def _():
        pltpu.sync_copy(tail_hbm, xt)
        tblock(xt, tt, D)
        pltpu.sync_copy(tt, out_hbm.at[pl.ds(V // 2 - 32, 32)])


def kernel(input_, weight):
    idx_flat = jnp.reshape(jnp.transpose(input_), (S * W,))
    wt = jnp.transpose(weight)
    wtail = jnp.transpose(lax.slice(weight, (V - D, 0), (V, D)))
    table2 = _pairs(wt, wtail)
    y = _emb(idx_flat, table2)
    return jnp.transpose(y, (2, 0, 1))


# speedup vs baseline: 2.4869x; 1.0020x over previous
import functools

import jax
import jax.numpy as jnp
from jax import lax
from jax.experimental import pallas as pl
from jax.experimental.pallas import tpu as pltpu
from jax.experimental.pallas import tpu_sc as plsc

V = 1000000
D = 64
S = 16384
W = 50
NW = 32
SB = S // 128
SB_W = SB // NW
NB = SB_W * W

_mesh = plsc.VectorSubcoreMesh(core_axis_name="c", subcore_axis_name="s")


@functools.partial(
    pl.kernel,
    out_type=jax.ShapeDtypeStruct((W, D, S), jnp.float32),
    mesh=_mesh,
    scratch_types=[
        pltpu.VMEM((4, 128), jnp.int32),
        pltpu.VMEM((4, 128), jnp.int32),
        pltpu.VMEM((4, 128), jnp.int32),
        pltpu.VMEM((4, 128, 128), jnp.float32),
        pltpu.VMEM((2, D, 128), jnp.float32),
        pltpu.SemaphoreType.DMA((4,)),
        pltpu.SemaphoreType.DMA((4,)),
        pltpu.SemaphoreType.DMA((2,)),
    ],
    compiler_params=pltpu.CompilerParams(
        use_tc_tiling_on_sc=True, needs_layout_passes=False
    ),
)
def _emb(idx_hbm, table_hbm, out_hbm, idxb, jdx, pcol, gbuf, tbuf,
         sem_g, sem_i, sem_w):
    wid = lax.axis_index("s") * 2 + lax.axis_index("c")
    sblock0 = wid * SB_W

    def idx_off(b):
        sb = b // W
        w = b - sb * W
        return pl.multiple_of(w * S + (sblock0 + sb) * 128, 128), w

    def fire_idx(b, s):
        off, _ = idx_off(b)
        pltpu.async_copy(idx_hbm.at[pl.ds(off, 128)], idxb.at[s], sem_i.at[s])

    def wait_idx(b, s):
        off, _ = idx_off(b)
        pltpu.make_async_copy(
            idx_hbm.at[pl.ds(off, 128)], idxb.at[s], sem_i.at[s]
        ).wait()

    def compute_jdx(s):
        for l0 in range(0, 128, 16):
            ids = idxb[s, pl.ds(l0, 16)]
            jdx[s, pl.ds(l0, 16)] = lax.shift_right_logical(ids, 1)
            pcol[s, pl.ds(l0, 16)] = (ids & jnp.int32(1)) * jnp.int32(D)

    def fire_gather(s):
        pltpu.async_copy(table_hbm.at[jdx.at[s]], gbuf.at[s], sem_g.at[s])

    def wait_gather(s):
        pltpu.make_async_copy(
            table_hbm.at[jdx.at[s]], gbuf.at[s], sem_g.at[s]
        ).wait()

    def transpose(s4, s2):
        lanes = lax.iota(jnp.int32, 16)

        def jstep(j, carry):
            rot = (lanes + j) & jnp.int32(15)
            rds = [rot + jnp.int32(d0) for d0 in range(0, D, 16)]
            for l0 in range(0, 128, 16):
                pcolv = pcol[s4, pl.ds(l0, 16)]
                rowsrc = lanes + jnp.int32(l0)
                for di in range(D // 16):
                    x = plsc.load_gather(
                        gbuf.at[s4], [rowsrc, pcolv + rds[di]]
                    )
                    plsc.store_scatter(tbuf.at[s2], [rds[di], rowsrc], x)
            return carry

        lax.fori_loop(0, 16, jstep, 0)

    def fire_wb(b, s):
        off, w = idx_off(b)
        s0 = pl.multiple_of(off - w * S, 128)
        pltpu.async_copy(
            tbuf.at[s], out_hbm.at[w, :, pl.ds(s0, 128)], sem_w.at[s]
        )

    def wait_wb(b, s):
        off, w = idx_off(b)
        s0 = pl.multiple_of(off - w * S, 128)
        pltpu.make_async_copy(
            tbuf.at[s], out_hbm.at[w, :, pl.ds(s0, 128)], sem_w.at[s]
        ).wait()

    fire_idx(0, 0)
    fire_idx(1, 1)
    fire_idx(2, 2)
    for b0 in range(3):
        wait_idx(b0, b0)
        compute_jdx(b0)
        fire_gather(b0)
    fire_idx(3, 3)

    def step(b, carry):
        s4 = lax.rem(b, 4)
        s2 = lax.rem(b, 2)
        wait_gather(s4)

        @pl.when(b + 3 < NB)
        def _():
            ns = lax.rem(b + 3, 4)
            wait_idx(b + 3, ns)
            compute_jdx(ns)
            fire_gather(ns)

        @pl.when(b + 4 < NB)
        def _():
            fire_idx(b + 4, s4)

        @pl.when(b >= 2)
        def _():
            wait_wb(b - 2, s2)

        transpose(s4, s2)
        fire_wb(b, s2)
        return carry

    lax.fori_loop(0, NB, step, 0)
    wait_wb(NB - 2, (NB - 2) % 2)
    wait_wb(NB - 1, (NB - 1) % 2)


NFB = V // 128


@functools.partial(
    pl.kernel,
    out_type=jax.ShapeDtypeStruct((V // 2, 2 * D), jnp.float32),
    mesh=_mesh,
    scratch_types=[
        pltpu.VMEM((3, D, 128), jnp.float32),
        pltpu.VMEM((2, D, 128), jnp.float32),
        pltpu.VMEM((D, D), jnp.float32),
        pltpu.VMEM((32, 2 * D), jnp.float32),
        pltpu.SemaphoreType.DMA((3,)),
        pltpu.SemaphoreType.DMA((2,)),
    ],
    compiler_params=pltpu.CompilerParams(
        use_tc_tiling_on_sc=True, needs_layout_passes=False
    ),
)
def _pairs(wt_hbm, tail_hbm, out_hbm, xb, tb, xt, tt, sem_i, sem_o):
    wid = lax.axis_index("s") * 2 + lax.axis_index("c")
    lanes = lax.iota(jnp.int32, 16)

    def blk(i):
        return i * NW + wid

    def fire_in(i, s):
        g = pl.multiple_of(blk(i) * 128, 128)
        pltpu.async_copy(wt_hbm.at[:, pl.ds(g, 128)], xb.at[s], sem_i.at[s])

    def wait_in(i, s):
        g = pl.multiple_of(blk(i) * 128, 128)
        pltpu.make_async_copy(
            wt_hbm.at[:, pl.ds(g, 128)], xb.at[s], sem_i.at[s]
        ).wait()

    def fire_out(i, s):
        r = pl.multiple_of(blk(i) * 64, 64)
        pltpu.async_copy(tb.at[s], out_hbm.at[pl.ds(r, 64)], sem_o.at[s])

    def wait_out(i, s):
        r = pl.multiple_of(blk(i) * 64, 64)
        pltpu.make_async_copy(
            tb.at[s], out_hbm.at[pl.ds(r, 64)], sem_o.at[s]
        ).wait()

    def tblock(src, dst, nt):
        half = (lanes & jnp.int32(1)) * jnp.int32(D)
        kh = lax.shift_right_logical(lanes, 1)

        def jstep(j, carry):
            rot = (lanes + j) & jnp.int32(15)
            srows = [rot + jnp.int32(d0) for d0 in range(0, D, 16)]
            dcols = [half + sr for sr in srows]
            for t0 in range(0, nt, 16):
                scols = lanes + jnp.int32(t0)
                drows = kh + jnp.int32(t0 // 2)
                for di in range(D // 16):
                    x = plsc.load_gather(src, [srows[di], scols])
                    plsc.store_scatter(dst, [drows, dcols[di]], x)
            return carry

        lax.fori_loop(0, 16, jstep, 0)

    NI = (NFB + NW - 1) // NW

    fire_in(0, 0)
    fire_in(1, 1)

    def step(i, carry):
        s3 = lax.rem(i, 3)
        s = lax.rem(i, 2)

        @pl.when(blk(i) < NFB)
        def _():
            wait_in(i, s3)

            @pl.when(blk(i + 2) < NFB)
            def _():
                fire_in(i + 2, lax.rem(i + 2, 3))

            @pl.when(i >= 2)
            def _():
                wait_out(i - 2, s)

            tblock(xb.at[s3], tb.at[s], 128)
            fire_out(i, s)

        return carry

    lax.fori_loop(0, NI, step, 0)

    @pl.when(blk(NI - 2) < NFB)
    def _():
        wait_out(NI - 2, (NI - 2) % 2)

    @pl.when(blk(NI - 1) < NFB)
    def _():
        wait_out(NI - 1, (NI - 1) % 2)

    @pl.when(wid == 0)
    def _():
        pltpu.sync_copy(tail_hbm, xt)
        tblock(xt, tt, D)
        pltpu.sync_copy(tt, out_hbm.at[pl.ds(V // 2 - 32, 32)])


def kernel(input_, weight):
    idx_flat = jnp.reshape(jnp.transpose(input_), (S * W,))
    wt = jnp.transpose(weight)
    wtail = jnp.transpose(lax.slice(weight, (V - D, 0), (V, D)))
    table2 = _pairs(wt, wtail)
    y = _emb(idx_flat, table2)
    return jnp.transpose(y, (2, 0, 1))
